# Initial kernel scaffold; baseline (speedup 1.0000x reference)
import jax
import jax.numpy as jnp
from jax.experimental import pallas as pl


def kernel(features, edge_index, W1, b1, W2, b2, W3, b3):
    raise NotImplementedError("write your pallas kernel here")


# speedup vs baseline: 4.5335x; 4.5335x over previous
import functools

import jax
import jax.numpy as jnp
from jax import lax
from jax.experimental import pallas as pl
from jax.experimental.pallas import tpu as pltpu
from jax.experimental.pallas import tpu_sc as plsc

NC = 2
NS = 16
NW = NC * NS
EB = 80
DW = 128
BLK = 400


@functools.lru_cache(maxsize=None)
def _make_deg(np_, e):
  e_per_w = e // NW
  n_chunks = e_per_w // EB
  rows_per_s = np_ // NS
  mesh = plsc.VectorSubcoreMesh(core_axis_name="c", subcore_axis_name="s")

  @functools.partial(
      pl.kernel,
      out_type=jax.ShapeDtypeStruct((NC, np_, DW), jnp.float32),
      mesh=mesh,
      scratch_types=[
          pltpu.VMEM((EB,), jnp.int32),
          pltpu.VMEM((EB, DW), jnp.float32),
          pltpu.VMEM_SHARED((np_, DW), jnp.float32),
      ],
  )
  def deg_kernel(dst_hbm, zeros_hbm, ones_hbm, out_hbm, dst_v, ones_v, acc_sh):
    cid = lax.axis_index("c")
    sid = lax.axis_index("s")
    r0 = sid * rows_per_s
    pltpu.sync_copy(zeros_hbm.at[pl.ds(r0, rows_per_s)],
                    acc_sh.at[pl.ds(r0, rows_per_s)])
    pltpu.sync_copy(ones_hbm, ones_v)
    plsc.subcore_barrier()
    ebase = (cid * NS + sid) * e_per_w

    def body(i, carry):
      off = ebase + i * EB
      pltpu.sync_copy(dst_hbm.at[pl.ds(off, EB)], dst_v)
      pltpu.sync_copy(ones_v, acc_sh.at[dst_v], add=True)
      return carry

    lax.fori_loop(0, n_chunks, body, 0)
    plsc.subcore_barrier()
    pltpu.sync_copy(acc_sh.at[pl.ds(r0, rows_per_s)],
                    out_hbm.at[cid, pl.ds(r0, rows_per_s)])

  return deg_kernel


@functools.lru_cache(maxsize=None)
def _make_agg(np_, e, d):
  e_per_w = e // NW
  n_chunks = e_per_w // EB
  rows_per_s = np_ // NS
  mesh = plsc.VectorSubcoreMesh(core_axis_name="c", subcore_axis_name="s")

  @functools.partial(
      pl.kernel,
      out_type=jax.ShapeDtypeStruct((NC, np_, d), jnp.float32),
      mesh=mesh,
      scratch_types=[
          pltpu.VMEM((EB,), jnp.int32),
          pltpu.VMEM((EB,), jnp.int32),
          pltpu.VMEM((EB, d), jnp.float32),
          pltpu.VMEM_SHARED((np_, d), jnp.float32),
          pltpu.SemaphoreType.DMA,
      ],
  )
  def agg_kernel(y_hbm, src_hbm, dst_hbm, zeros_hbm, out_hbm,
                 src_v, dst_v, rows_v, acc_sh, sem):
    cid = lax.axis_index("c")
    sid = lax.axis_index("s")
    r0 = sid * rows_per_s
    pltpu.sync_copy(zeros_hbm.at[pl.ds(r0, rows_per_s)],
                    acc_sh.at[pl.ds(r0, rows_per_s)])
    plsc.subcore_barrier()
    ebase = (cid * NS + sid) * e_per_w

    def body(i, carry):
      off = ebase + i * EB
      pltpu.sync_copy(src_hbm.at[pl.ds(off, EB)], src_v)
      pltpu.sync_copy(dst_hbm.at[pl.ds(off, EB)], dst_v)
      pltpu.async_copy(y_hbm.at[src_v], rows_v, sem).wait()
      pltpu.sync_copy(rows_v, acc_sh.at[dst_v], add=True)
      return carry

    lax.fori_loop(0, n_chunks, body, 0)
    plsc.subcore_barrier()
    pltpu.sync_copy(acc_sh.at[pl.ds(r0, rows_per_s)],
                    out_hbm.at[cid, pl.ds(r0, rows_per_s)])

  return agg_kernel


def _tc_layer0(x, w, deg_p):
  n, d_in = x.shape
  d_h = w.shape[1]

  def body(x_ref, w_ref, deg_ref, y_ref, dinv_ref):
    deg = (deg_ref[0] + deg_ref[1])[:, :1]
    dinv = jnp.where(deg > 0, lax.rsqrt(jnp.maximum(deg, 1.0)), 0.0)
    dinv_ref[...] = dinv
    t = jnp.dot(x_ref[...], w_ref[...], preferred_element_type=jnp.float32)
    y_ref[...] = t * dinv

  return pl.pallas_call(
      body,
      grid=(n // BLK,),
      in_specs=[
          pl.BlockSpec((BLK, d_in), lambda i: (i, 0)),
          pl.BlockSpec((d_in, d_h), lambda i: (0, 0)),
          pl.BlockSpec((2, BLK, DW), lambda i: (0, i, 0)),
      ],
      out_specs=[
          pl.BlockSpec((BLK, d_h), lambda i: (i, 0)),
          pl.BlockSpec((BLK, 1), lambda i: (i, 0)),
      ],
      out_shape=[
          jax.ShapeDtypeStruct((n, d_h), jnp.float32),
          jax.ShapeDtypeStruct((n, 1), jnp.float32),
      ],
  )(x, w, deg_p)


def _tc_mid(p, dinv, b, w):
  n = dinv.shape[0]
  d = p.shape[2]
  d2 = w.shape[1]

  def body(p_ref, dinv_ref, b_ref, w_ref, y_ref):
    s = p_ref[0] + p_ref[1]
    dv = dinv_ref[...]
    h = jnp.maximum(s * dv + b_ref[...], 0.0)
    y_ref[...] = jnp.dot(h, w_ref[...], preferred_element_type=jnp.float32) * dv

  return pl.pallas_call(
      body,
      grid=(n // BLK,),
      in_specs=[
          pl.BlockSpec((2, BLK, d), lambda i: (0, i, 0)),
          pl.BlockSpec((BLK, 1), lambda i: (i, 0)),
          pl.BlockSpec((1, d), lambda i: (0, 0)),
          pl.BlockSpec((d, d2), lambda i: (0, 0)),
      ],
      out_specs=pl.BlockSpec((BLK, d2), lambda i: (i, 0)),
      out_shape=jax.ShapeDtypeStruct((n, d2), jnp.float32),
  )(p, dinv, b, w)


def _tc_mid_nomm(p, dinv, b):
  n = dinv.shape[0]
  d = p.shape[2]

  def body(p_ref, dinv_ref, b_ref, y_ref):
    s = p_ref[0] + p_ref[1]
    dv = dinv_ref[...]
    y_ref[...] = jnp.maximum(s * dv + b_ref[...], 0.0) * dv

  return pl.pallas_call(
      body,
      grid=(n // BLK,),
      in_specs=[
          pl.BlockSpec((2, BLK, d), lambda i: (0, i, 0)),
          pl.BlockSpec((BLK, 1), lambda i: (i, 0)),
          pl.BlockSpec((1, d), lambda i: (0, 0)),
      ],
      out_specs=pl.BlockSpec((BLK, d), lambda i: (i, 0)),
      out_shape=jax.ShapeDtypeStruct((n, d), jnp.float32),
  )(p, dinv, b)


def _tc_final(p, dinv, b, w):
  n = dinv.shape[0]
  d = p.shape[2]
  d2 = w.shape[1]

  def body(p_ref, dinv_ref, b_ref, w_ref, y_ref):
    s = (p_ref[0] + p_ref[1]) * dinv_ref[...]
    y_ref[...] = jnp.dot(s, w_ref[...],
                         preferred_element_type=jnp.float32) + b_ref[...]

  return pl.pallas_call(
      body,
      grid=(n // BLK,),
      in_specs=[
          pl.BlockSpec((2, BLK, d), lambda i: (0, i, 0)),
          pl.BlockSpec((BLK, 1), lambda i: (i, 0)),
          pl.BlockSpec((1, d2), lambda i: (0, 0)),
          pl.BlockSpec((d, d2), lambda i: (0, 0)),
      ],
      out_specs=pl.BlockSpec((BLK, d2), lambda i: (i, 0)),
      out_shape=jax.ShapeDtypeStruct((n, d2), jnp.float32),
  )(p, dinv, b, w)


def kernel(features, edge_index, W1, b1, W2, b2, W3, b3):
  n, _ = features.shape
  e = edge_index.shape[1]
  d_h = W1.shape[1]
  d_out = W3.shape[1]
  src = edge_index[0]
  dst = edge_index[1]
  np_ = ((n + NS * 8 - 1) // (NS * 8)) * (NS * 8)

  zeros_h = jnp.zeros((np_, d_h), jnp.float32)
  zeros_dw = jnp.zeros((np_, DW), jnp.float32)
  ones_eb = jnp.ones((EB, DW), jnp.float32)

  deg_p = _make_deg(np_, e)(dst, zeros_dw, ones_eb)

  y1, dinv = _tc_layer0(features, W1, deg_p)
  p = _make_agg(np_, e, d_h)(y1, src, dst, zeros_h)
  y2 = _tc_mid(p, dinv, b1.reshape(1, -1), W2)
  p = _make_agg(np_, e, d_h)(y2, src, dst, zeros_h)
  y3 = _tc_mid_nomm(p, dinv, b2.reshape(1, -1))
  p = _make_agg(np_, e, d_h)(y3, src, dst, zeros_h)
  return _tc_final(p, dinv, b3.reshape(1, -1), W3)
